# baseline (device time: 339307 ns/iter reference)
import jax
import jax.numpy as jnp
from jax import lax
from jax.experimental import pallas as pl
from jax.experimental.pallas import tpu as pltpu

N_DEV = 16
SQ = 2048
D_MODEL = 1024
H_PER = 8
DH = 128
H_SLICE = H_PER * DH
CHUNK = SQ // N_DEV
BLK = 64
SCALE = 0.08838834764831843

N_STEP = N_DEV - 1


def _allreduce_body(x_ref, out_ref, g_ref, rb0, rb1, rb2, rb3,
                    rs_send_sems, rs_recv_sems, ag_send_sems, ag_recv_sems):
    pos = lax.axis_index("i")
    w = lax.rem(pos, 4)
    z = pos // 4
    b_x = jnp.where((w == 1) | (w == 2), 1, 0).astype(jnp.int32)
    b_y = w // 2
    b_z0 = lax.rem(z, 2)
    b_z1 = z // 2
    p_x = pos + 1 - 2 * lax.rem(w, 2)
    p_y = pos + 3 - 2 * w
    p_z0 = pos + (1 - 2 * b_z0) * 4
    p_z1 = pos + (1 - 2 * b_z1) * 8

    out_ref[...] = x_ref[...]

    active = jnp.int32(0)
    rs_steps = [(8, b_x, p_x, rb0), (4, b_y, p_y, rb1),
                (2, b_z0, p_z0, rb2), (1, b_z1, p_z1, rb3)]
    for k, (h, b, partner, rbuf) in enumerate(rs_steps):
        keep = active + b * h
        send = active + (1 - b) * h
        s_sl = pl.ds(send * CHUNK, h * CHUNK)
        g_ref[s_sl, :] = out_ref[s_sl, :].astype(jnp.bfloat16)
        rdma = pltpu.make_async_remote_copy(
            src_ref=g_ref.at[s_sl, :],
            dst_ref=rbuf,
            send_sem=rs_send_sems.at[k],
            recv_sem=rs_recv_sems.at[k],
            device_id=(partner,),
            device_id_type=pl.DeviceIdType.MESH,
        )
        rdma.start()
        rdma.wait()
        k_sl = pl.ds(keep * CHUNK, h * CHUNK)
        out_ref[k_sl, :] = out_ref[k_sl, :] + rbuf[...].astype(jnp.float32)
        active = keep

    own_sl = pl.ds(active * CHUNK, CHUNK)
    g_ref[own_sl, :] = out_ref[own_sl, :].astype(jnp.bfloat16)

    ag_steps = [(1, b_z1, p_z1), (2, b_z0, p_z0),
                (4, b_y, p_y), (8, b_x, p_x)]
    for k, (sz, b, partner) in enumerate(ag_steps):
        new = active - b * sz
        other = new + (1 - b) * sz
        my_sl = pl.ds(active * CHUNK, sz * CHUNK)
        rdma = pltpu.make_async_remote_copy(
            src_ref=g_ref.at[my_sl, :],
            dst_ref=g_ref.at[my_sl, :],
            send_sem=ag_send_sems.at[k],
            recv_sem=ag_recv_sems.at[k],
            device_id=(partner,),
            device_id_type=pl.DeviceIdType.MESH,
        )
        rdma.start()
        rdma.wait()
        o_sl = pl.ds(other * CHUNK, sz * CHUNK)
        out_ref[o_sl, :] = g_ref[o_sl, :].astype(jnp.float32)
        active = new


def _ring_allreduce(partial):
    return pl.pallas_call(
        _allreduce_body,
        out_shape=jax.ShapeDtypeStruct((SQ, D_MODEL), jnp.float32),
        in_specs=[pl.BlockSpec(memory_space=pltpu.VMEM)],
        out_specs=pl.BlockSpec(memory_space=pltpu.VMEM),
        scratch_shapes=[
            pltpu.VMEM((SQ, D_MODEL), jnp.bfloat16),
            pltpu.VMEM((8 * CHUNK, D_MODEL), jnp.bfloat16),
            pltpu.VMEM((4 * CHUNK, D_MODEL), jnp.bfloat16),
            pltpu.VMEM((2 * CHUNK, D_MODEL), jnp.bfloat16),
            pltpu.VMEM((1 * CHUNK, D_MODEL), jnp.bfloat16),
            pltpu.SemaphoreType.DMA((4,)),
            pltpu.SemaphoreType.DMA((4,)),
            pltpu.SemaphoreType.DMA((4,)),
            pltpu.SemaphoreType.DMA((4,)),
        ],
    )(partial)


QT = 256
NQ = SQ // QT


def _attn_body(x_ref, wq_ref, k_ref, v_ref, wo_ref, out_ref):
    i = pl.program_id(0)
    h = pl.program_id(1)

    q = jnp.dot(x_ref[...], wq_ref[...],
                preferred_element_type=jnp.float32)
    q = (q * SCALE).astype(jnp.bfloat16)

    def tile(j, m, l, acc, masked):
        kj = k_ref[pl.ds(j * QT, QT), :]
        s = lax.dot_general(q, kj, (((1,), (1,)), ((), ())),
                            preferred_element_type=jnp.float32)
        if masked:
            r = lax.broadcasted_iota(jnp.int32, (QT, QT), 0) // BLK
            c = lax.broadcasted_iota(jnp.int32, (QT, QT), 1) // BLK
            s = jnp.where(c <= r, s, -1e9)
        mj = jnp.maximum(m, jnp.max(s, axis=1, keepdims=True))
        p = jnp.exp(s - mj)
        corr = jnp.exp(m - mj)
        l = l * corr + jnp.sum(p, axis=1, keepdims=True)
        vj = v_ref[pl.ds(j * QT, QT), :]
        pv = lax.dot_general(p.astype(jnp.bfloat16), vj,
                             (((1,), (0,)), ((), ())),
                             preferred_element_type=jnp.float32)
        acc = acc * corr + pv
        return mj, l, acc

    m0 = jnp.full((QT, 1), -1e30, jnp.float32)
    l0 = jnp.zeros((QT, 1), jnp.float32)
    a0 = jnp.zeros((QT, DH), jnp.float32)
    m, l, acc = lax.fori_loop(
        0, i, lambda j, c: tile(j, *c, masked=False), (m0, l0, a0))
    m, l, acc = tile(i, m, l, acc, masked=True)

    o = (acc / l).astype(jnp.bfloat16)
    contrib = jnp.dot(o, wo_ref[...],
                      preferred_element_type=jnp.float32)

    @pl.when(h == 0)
    def _():
        out_ref[...] = contrib

    @pl.when(h > 0)
    def _():
        out_ref[...] = out_ref[...] + contrib


def _fused_partial(x, wq_l, k, v, wo_l):
    return pl.pallas_call(
        _attn_body,
        out_shape=jax.ShapeDtypeStruct((SQ, D_MODEL), jnp.float32),
        grid=(NQ, H_PER),
        in_specs=[
            pl.BlockSpec((QT, D_MODEL), lambda i, h: (i, 0)),
            pl.BlockSpec((D_MODEL, DH), lambda i, h: (0, h)),
            pl.BlockSpec((SQ, DH), lambda i, h: (0, h)),
            pl.BlockSpec((SQ, DH), lambda i, h: (0, h)),
            pl.BlockSpec((DH, D_MODEL), lambda i, h: (h, 0)),
        ],
        out_specs=pl.BlockSpec((QT, D_MODEL), lambda i, h: (i, 0)),
    )(x, wq_l, k, v, wo_l)


def kernel(x, Wq, K_ext, V_ext, Wo):
    pos = lax.axis_index("i")
    bf = jnp.bfloat16

    x2 = x[0].astype(bf)
    Wq_l = lax.dynamic_slice_in_dim(Wq, pos * H_SLICE, H_SLICE,
                                    axis=1).astype(bf)
    Wo_l = lax.dynamic_slice_in_dim(Wo, pos * H_SLICE, H_SLICE,
                                    axis=0).astype(bf)
    k = K_ext[0].reshape(SQ, H_SLICE).astype(bf)
    v = V_ext[0].reshape(SQ, H_SLICE).astype(bf)

    partial = _fused_partial(x2, Wq_l, k, v, Wo_l)
    out = _ring_allreduce(partial)
    return out[None, :, :]
